# baseline (device time: 12966 ns/iter reference)
import jax
import jax.numpy as jnp
from jax import lax
from jax.experimental import pallas as pl
from jax.experimental.pallas import tpu as pltpu

BLOCK_M = 512


def kernel(x):
    m, n = x.shape
    n_blocks = m // BLOCK_M
    g = m // 128

    def body(x_ref, out_ref, acc_ref, recv_ref, send_sem, recv_sem):
        i = pl.program_id(0)
        my_x = lax.axis_index("x")
        my_y = lax.axis_index("y")
        nbr = (my_x, 1 - my_y)

        barrier_sem = pltpu.get_barrier_semaphore()

        @pl.when(i == 0)
        def _():
            pl.semaphore_signal(
                barrier_sem, inc=1,
                device_id=nbr, device_id_type=pl.DeviceIdType.MESH,
            )

        acc_ref[pl.ds(i * BLOCK_M, BLOCK_M), :] = jnp.sum(
            x_ref[:, :], axis=1, keepdims=True
        )

        @pl.when(i == n_blocks - 1)
        def _():
            out_ref[:, :] = acc_ref[:, :].reshape(g, 128)

            pl.semaphore_wait(barrier_sem, 1)

            rdma = pltpu.make_async_remote_copy(
                src_ref=out_ref,
                dst_ref=recv_ref,
                send_sem=send_sem,
                recv_sem=recv_sem,
                device_id=nbr,
                device_id_type=pl.DeviceIdType.MESH,
            )
            rdma.start()
            rdma.wait()

            out_ref[:, :] = out_ref[:, :] + recv_ref[:, :]

    x = pltpu.with_memory_space_constraint(x, pltpu.MemorySpace.HBM)

    out = pl.pallas_call(
        body,
        grid=(n_blocks,),
        out_shape=jax.ShapeDtypeStruct((g, 128), jnp.float32),
        in_specs=[
            pl.BlockSpec((BLOCK_M, n), lambda i: (i, 0), memory_space=pltpu.VMEM)
        ],
        out_specs=pl.BlockSpec((g, 128), lambda i: (0, 0), memory_space=pltpu.VMEM),
        scratch_shapes=[
            pltpu.VMEM((m, 1), jnp.float32),
            pltpu.VMEM((g, 128), jnp.float32),
            pltpu.SemaphoreType.DMA,
            pltpu.SemaphoreType.DMA,
        ],
        compiler_params=pltpu.CompilerParams(collective_id=0),
    )(x)
    return out.reshape(m, 1)


# device time: 10315 ns/iter; 1.2570x vs baseline; 1.2570x over previous
import jax
import jax.numpy as jnp
from jax import lax
from jax.experimental import pallas as pl
from jax.experimental.pallas import tpu as pltpu

BLOCK_M = 512


def kernel(x):
    m, n = x.shape
    nb = m // BLOCK_M
    g = m // 128
    gb = BLOCK_M // 128

    def body(x_hbm, out_ref, buf, recv_ref, copy_sems, send_sem, recv_sem):
        my_x = lax.axis_index("x")
        my_y = lax.axis_index("y")
        nbr = (my_x, 1 - my_y)

        barrier_sem = pltpu.get_barrier_semaphore()
        pl.semaphore_signal(
            barrier_sem, inc=1,
            device_id=nbr, device_id_type=pl.DeviceIdType.MESH,
        )

        copies = [
            pltpu.make_async_copy(
                x_hbm.at[pl.ds(i * BLOCK_M, BLOCK_M), :],
                buf.at[i],
                copy_sems.at[i],
            )
            for i in range(nb)
        ]
        for c in copies:
            c.start()
        for i, c in enumerate(copies):
            c.wait()
            part = jnp.sum(buf[i], axis=1, keepdims=True)
            out_ref[pl.ds(i * gb, gb), :] = part.reshape(gb, 128)

        pl.semaphore_wait(barrier_sem, 1)

        rdma = pltpu.make_async_remote_copy(
            src_ref=out_ref,
            dst_ref=recv_ref,
            send_sem=send_sem,
            recv_sem=recv_sem,
            device_id=nbr,
            device_id_type=pl.DeviceIdType.MESH,
        )
        rdma.start()
        rdma.wait()

        out_ref[:, :] = out_ref[:, :] + recv_ref[:, :]

    x = pltpu.with_memory_space_constraint(x, pltpu.MemorySpace.HBM)

    out = pl.pallas_call(
        body,
        out_shape=jax.ShapeDtypeStruct((g, 128), jnp.float32),
        in_specs=[pl.BlockSpec(memory_space=pl.ANY)],
        out_specs=pl.BlockSpec(memory_space=pltpu.VMEM),
        scratch_shapes=[
            pltpu.VMEM((nb, BLOCK_M, n), jnp.float32),
            pltpu.VMEM((g, 128), jnp.float32),
            pltpu.SemaphoreType.DMA((nb,)),
            pltpu.SemaphoreType.DMA,
            pltpu.SemaphoreType.DMA,
        ],
        compiler_params=pltpu.CompilerParams(collective_id=0),
    )(x)
    return out.reshape(m, 1)


# device time: 10304 ns/iter; 1.2583x vs baseline; 1.0011x over previous
import jax
import jax.numpy as jnp
from jax import lax
from jax.experimental import pallas as pl
from jax.experimental.pallas import tpu as pltpu

BLOCK_M = 512


def kernel(x):
    m, n = x.shape
    nb = m // BLOCK_M
    g = m // 128
    gb = BLOCK_M // 128

    def body(x_hbm, out_ref, buf, recv_ref, copy_sems, send_sem, recv_sem):
        my_x = lax.axis_index("x")
        my_y = lax.axis_index("y")
        nbr = (my_x, 1 - my_y)

        barrier_sem = pltpu.get_barrier_semaphore()
        pl.semaphore_signal(
            barrier_sem, inc=1,
            device_id=nbr, device_id_type=pl.DeviceIdType.MESH,
        )

        copies = [
            pltpu.make_async_copy(
                x_hbm.at[pl.ds(i * BLOCK_M, BLOCK_M), :],
                buf.at[i],
                copy_sems.at[i],
            )
            for i in range(nb)
        ]
        for c in copies:
            c.start()
        for i, c in enumerate(copies):
            c.wait()
            part = jnp.sum(buf[i], axis=1, keepdims=True)
            out_ref[pl.ds(i * gb, gb), :] = part.reshape(gb, 128)

        pl.semaphore_wait(barrier_sem, 1)

        rdma = pltpu.make_async_remote_copy(
            src_ref=out_ref,
            dst_ref=recv_ref,
            send_sem=send_sem,
            recv_sem=recv_sem,
            device_id=nbr,
            device_id_type=pl.DeviceIdType.MESH,
        )
        rdma.start()
        rdma.wait()

        out_ref[:, :] = out_ref[:, :] + recv_ref[:, :]

    x = pltpu.with_memory_space_constraint(x, pltpu.MemorySpace.HBM)

    out = pl.pallas_call(
        body,
        out_shape=jax.ShapeDtypeStruct((g, 128), jnp.float32),
        in_specs=[pl.BlockSpec(memory_space=pl.ANY)],
        out_specs=pl.BlockSpec(memory_space=pltpu.VMEM),
        scratch_shapes=[
            pltpu.VMEM((nb, BLOCK_M, n), jnp.float32),
            pltpu.VMEM((g, 128), jnp.float32),
            pltpu.SemaphoreType.DMA((nb,)),
            pltpu.SemaphoreType.DMA,
            pltpu.SemaphoreType.DMA,
        ],
        compiler_params=pltpu.CompilerParams(
            collective_id=0, disable_bounds_checks=True
        ),
    )(x)
    return out.reshape(m, 1)
